# baseline (speedup 1.0000x reference)
import functools

import jax
import jax.numpy as jnp
from jax import lax
from jax.experimental import pallas as pl
from jax.experimental.pallas import tpu as pltpu
from jax.experimental.pallas import tpu_sc as plsc

_N = 10000
_D = 256
_E = 160000
_CH = 64
_NCH = 160
_EPT = _CH * _NCH
_EP = 16 * _EPT
_AGG_ROWS = 10240
_ZCOPIES = _AGG_ROWS // (16 * _CH)
_ORT = _N // 16
_NBUF = 4
_GAHEAD = 2
_NHALF = 4
_HCH = _NCH // _NHALF


@functools.cache
def _make_segsum():
    mesh = plsc.VectorSubcoreMesh(core_axis_name="c", subcore_axis_name="s")

    @functools.partial(
        pl.kernel,
        mesh=mesh,
        out_type=jax.ShapeDtypeStruct((_N, 2, 128), jnp.float32),
        scratch_types=[
            pltpu.VMEM((_HCH, _CH), jnp.int32),
            pltpu.VMEM((_HCH, _CH), jnp.int32),
        ]
        + [pltpu.VMEM((_CH, 128), jnp.float32) for _ in range(_NBUF)]
        + [pltpu.VMEM_SHARED((_AGG_ROWS, 128), jnp.float32)]
        + [pltpu.SemaphoreType.DMA for _ in range(2 * _NBUF)],
    )
    def segsum(tab, src2, dstp, out, src_v, dst_v, *rest):
        rows = rest[:_NBUF]
        agg_sh = rest[_NBUF]
        gsem = rest[_NBUF + 1:_NBUF + 1 + _NBUF]
        ssem = rest[_NBUF + 1 + _NBUF:]
        c = lax.axis_index("c")
        s = lax.axis_index("s")

        def zb(i, carry):
            for q in range(8):
                rows[0][i, pl.ds(q * 16, 16)] = jnp.zeros((16,), jnp.float32)
            return carry

        lax.fori_loop(0, _CH, zb, 0)
        for k in range(_ZCOPIES):
            pltpu.sync_copy(
                rows[0], agg_sh.at[pl.ds(s * (_ZCOPIES * _CH) + k * _CH, _CH)]
            )
        plsc.subcore_barrier()

        def gissue(j, b):
            pltpu.async_copy(tab.at[src_v.at[j]], rows[b], gsem[b])

        def gwait(j, b):
            pltpu.make_async_copy(tab.at[src_v.at[j]], rows[b], gsem[b]).wait()

        def sissue(j, b):
            pltpu.async_copy(rows[b], agg_sh.at[dst_v.at[j]], ssem[b], add=True)

        def swait(j, b):
            pltpu.make_async_copy(
                rows[b], agg_sh.at[dst_v.at[j]], ssem[b]
            ).wait()

        for h in range(_NHALF):
            pltpu.sync_copy(src2.at[c, s, pl.ds(h * _HCH, _HCH)], src_v)
            pltpu.sync_copy(dstp.at[s, pl.ds(h * _HCH, _HCH)], dst_v)

            for k0 in range(_GAHEAD):
                gissue(k0, k0 % _NBUF)

            def body(g, carry):
                for b in range(_NBUF):
                    k = g * _NBUF + b
                    gwait(k, b)
                    sissue(k, b)
                    kn = k + _GAHEAD
                    bn = (b + _GAHEAD) % _NBUF

                    @pl.when(kn < _HCH)
                    def _():
                        @pl.when(kn >= _NBUF)
                        def _():
                            swait(kn - _NBUF, bn)

                        gissue(kn, bn)

                return carry

            lax.fori_loop(0, _HCH // _NBUF, body, 0)
            for m in range(_HCH - _NBUF, _HCH):
                swait(m, m % _NBUF)
        plsc.subcore_barrier()

        base = s * _ORT
        pltpu.sync_copy(agg_sh.at[pl.ds(base, _ORT)], out.at[pl.ds(base, _ORT), c])

    return segsum


def _mk_mlp_body(with_res):
    def body(*refs):
        if with_res:
            x_ref, a_ref, r_ref, w1_ref, b1_ref, w2_ref, b2_ref, u_ref, st_ref = refs
        else:
            x_ref, a_ref, w1_ref, b1_ref, w2_ref, b2_ref, u_ref, st_ref = refs
        t = x_ref[...] + a_ref[...]
        m = jnp.maximum(
            jnp.dot(t, w1_ref[...], preferred_element_type=jnp.float32) + b1_ref[...],
            0.0,
        )
        u = jnp.dot(m, w2_ref[...], preferred_element_type=jnp.float32) + b2_ref[...]
        if with_res:
            u = u + r_ref[...]
        u_ref[...] = u
        su = jnp.sum(u, axis=0, keepdims=True)
        sq = jnp.sum(u * u, axis=0, keepdims=True)
        acc = jnp.concatenate(
            [su, sq, jnp.zeros((6, u.shape[1]), jnp.float32)], axis=0
        )

        @pl.when(pl.program_id(0) == 0)
        def _():
            st_ref[...] = acc

        @pl.when(pl.program_id(0) != 0)
        def _():
            st_ref[...] = st_ref[...] + acc

    return body


_BR = 1000


def _mlp(x, agg, res, W1, b1, W2, b2):
    grid = (_N // _BR,)
    row_spec = pl.BlockSpec((_BR, _D), lambda i: (i, 0))
    full_spec = pl.BlockSpec((_D, _D), lambda i: (0, 0))
    vec_spec = pl.BlockSpec((1, _D), lambda i: (0, 0))
    st_spec = pl.BlockSpec((8, _D), lambda i: (0, 0))
    with_res = res is not None
    ins = [x, agg] + ([res] if with_res else [])
    ins += [W1, b1.reshape(1, _D), W2, b2.reshape(1, _D)]
    in_specs = [row_spec, row_spec] + ([row_spec] if with_res else [])
    in_specs += [full_spec, vec_spec, full_spec, vec_spec]
    return pl.pallas_call(
        _mk_mlp_body(with_res),
        grid=grid,
        in_specs=in_specs,
        out_specs=[row_spec, st_spec],
        out_shape=[
            jax.ShapeDtypeStruct((_N, _D), jnp.float32),
            jax.ShapeDtypeStruct((8, _D), jnp.float32),
        ],
    )(*ins)


def _bn_body(u_ref, st_ref, g_ref, b_ref, o_ref):
    inv_n = jnp.float32(1.0 / _N)
    mean = st_ref[0:1, :] * inv_n
    var = st_ref[1:2, :] * inv_n - mean * mean
    scale = g_ref[...] * lax.rsqrt(var + 1e-5)
    shift = b_ref[...] - mean * scale
    o_ref[...] = jnp.maximum(u_ref[...] * scale + shift, 0.0)


def _bn_relu(u, st, g, be):
    grid = (_N // _BR,)
    row_spec = pl.BlockSpec((_BR, _D), lambda i: (i, 0))
    st_spec = pl.BlockSpec((8, _D), lambda i: (0, 0))
    vec_spec = pl.BlockSpec((1, _D), lambda i: (0, 0))
    return pl.pallas_call(
        _bn_body,
        grid=grid,
        in_specs=[row_spec, st_spec, vec_spec, vec_spec],
        out_specs=row_spec,
        out_shape=jax.ShapeDtypeStruct((_N, _D), jnp.float32),
    )(u, st, g.reshape(1, _D), be.reshape(1, _D))


def _heads_body(h_ref, a_ref, w1m, b1m, w2m, b2m, w1l, b1l, w2l, b2l, mu_ref, lv_ref):
    t = h_ref[...] + a_ref[...]

    def head(w1, b1, w2, b2, o_ref):
        m = jnp.maximum(
            jnp.dot(t, w1[...], preferred_element_type=jnp.float32) + b1[...], 0.0
        )
        z = jnp.dot(m, w2[...], preferred_element_type=jnp.float32) + b2[...]
        o_ref[...] = 1.0 / (1.0 + jnp.exp(-z))

    head(w1m, b1m, w2m, b2m, mu_ref)
    head(w1l, b1l, w2l, b2l, lv_ref)


def _heads(h, agg, W1m, b1m, W2m, b2m, W1l, b1l, W2l, b2l):
    grid = (_N // _BR,)
    row_spec = pl.BlockSpec((_BR, _D), lambda i: (i, 0))
    full_spec = pl.BlockSpec((_D, _D), lambda i: (0, 0))
    vec_spec = pl.BlockSpec((1, _D), lambda i: (0, 0))
    return pl.pallas_call(
        _heads_body,
        grid=grid,
        in_specs=[row_spec, row_spec] + [full_spec, vec_spec] * 4,
        out_specs=[row_spec, row_spec],
        out_shape=[
            jax.ShapeDtypeStruct((_N, _D), jnp.float32),
            jax.ShapeDtypeStruct((_N, _D), jnp.float32),
        ],
    )(
        h, agg,
        W1m, b1m.reshape(1, _D), W2m, b2m.reshape(1, _D),
        W1l, b1l.reshape(1, _D), W2l, b2l.reshape(1, _D),
    )


def kernel(x, edge_index,
           W1_0, b1_0, W2_0, b2_0,
           W1_1, b1_1, W2_1, b2_1,
           W1_mu, b1_mu, W2_mu, b2_mu,
           W1_lv, b1_lv, W2_lv, b2_lv,
           g0, be0, g1, be1):
    src = edge_index[0].astype(jnp.int32)
    dst = edge_index[1].astype(jnp.int32)
    pad = jnp.full((_EP - _E,), _N, jnp.int32)
    srcp = jnp.concatenate([src, pad])
    dstp = jnp.concatenate([dst, pad]).reshape(16, _NCH, _CH)
    src2 = jnp.stack([2 * srcp, 2 * srcp + 1]).reshape(2, 16, _NCH, _CH)
    zrow = jnp.zeros((1, _D), jnp.float32)

    def conv_agg(h):
        tab = jnp.concatenate([h, zrow], axis=0).reshape(2 * (_N + 1), 128)
        return _make_segsum()(tab, src2, dstp).reshape(_N, _D)

    agg0 = conv_agg(x)
    u0, st0 = _mlp(x, agg0, None, W1_0, b1_0, W2_0, b2_0)
    h0 = _bn_relu(u0, st0, g0, be0)

    agg1 = conv_agg(h0)
    u1, st1 = _mlp(h0, agg1, h0, W1_1, b1_1, W2_1, b2_1)
    h1 = _bn_relu(u1, st1, g1, be1)

    agg2 = conv_agg(h1)
    mu, lv = _heads(h1, agg2,
                    W1_mu, b1_mu, W2_mu, b2_mu,
                    W1_lv, b1_lv, W2_lv, b2_lv)
    return (mu, lv)


# speedup vs baseline: 1.1345x; 1.1345x over previous
import functools

import jax
import jax.numpy as jnp
from jax import lax
from jax.experimental import pallas as pl
from jax.experimental.pallas import tpu as pltpu
from jax.experimental.pallas import tpu_sc as plsc

_N = 10000
_D = 256
_E = 160000
_CH = 128
_NCH = 80
_EPT = _CH * _NCH
_EP = 16 * _EPT
_AGG_ROWS = 10240
_ZCOPIES = _AGG_ROWS // (16 * _CH)
_ORT = _N // 16
_NBUF = 2
_NHALF = 2
_HCH = _NCH // _NHALF


@functools.cache
def _make_segsum():
    mesh = plsc.VectorSubcoreMesh(core_axis_name="c", subcore_axis_name="s")

    @functools.partial(
        pl.kernel,
        mesh=mesh,
        out_type=jax.ShapeDtypeStruct((_N, 2, 128), jnp.float32),
        scratch_types=[
            pltpu.VMEM((_HCH, _CH), jnp.int32),
            pltpu.VMEM((_HCH, _CH), jnp.int32),
        ]
        + [pltpu.VMEM((_CH, 128), jnp.float32) for _ in range(_NBUF)]
        + [pltpu.VMEM_SHARED((_AGG_ROWS, 128), jnp.float32)]
        + [pltpu.SemaphoreType.DMA for _ in range(_NBUF)],
    )
    def segsum(tab, src2, dstp, out, src_v, dst_v, *rest):
        rows = rest[:_NBUF]
        agg_sh = rest[_NBUF]
        gsem = rest[_NBUF + 1:]
        c = lax.axis_index("c")
        s = lax.axis_index("s")

        def zb(i, carry):
            for q in range(8):
                rows[0][i, pl.ds(q * 16, 16)] = jnp.zeros((16,), jnp.float32)
            return carry

        lax.fori_loop(0, _CH, zb, 0)
        for k in range(_ZCOPIES):
            pltpu.sync_copy(
                rows[0], agg_sh.at[pl.ds(s * (_ZCOPIES * _CH) + k * _CH, _CH)]
            )
        plsc.subcore_barrier()

        for h in range(_NHALF):
            pltpu.sync_copy(src2.at[c, s, pl.ds(h * _HCH, _HCH)], src_v)
            pltpu.sync_copy(dstp.at[s, pl.ds(h * _HCH, _HCH)], dst_v)

            for b in range(_NBUF):
                pltpu.async_copy(tab.at[src_v.at[b]], rows[b], gsem[b])

            def body(g, carry):
                for b in range(_NBUF):
                    j = g * _NBUF + b
                    pltpu.make_async_copy(
                        tab.at[src_v.at[j]], rows[b], gsem[b]
                    ).wait()
                    pltpu.sync_copy(rows[b], agg_sh.at[dst_v.at[j]], add=True)
                    jn = j + _NBUF

                    @pl.when(jn < _HCH)
                    def _():
                        pltpu.async_copy(tab.at[src_v.at[jn]], rows[b], gsem[b])

                return carry

            lax.fori_loop(0, _HCH // _NBUF, body, 0)
        plsc.subcore_barrier()

        base = s * _ORT
        pltpu.sync_copy(agg_sh.at[pl.ds(base, _ORT)], out.at[pl.ds(base, _ORT), c])

    return segsum


def _mk_mlp_body(with_res):
    def body(*refs):
        if with_res:
            x_ref, a_ref, r_ref, w1_ref, b1_ref, w2_ref, b2_ref, u_ref, st_ref = refs
        else:
            x_ref, a_ref, w1_ref, b1_ref, w2_ref, b2_ref, u_ref, st_ref = refs
        t = x_ref[...] + a_ref[...]
        m = jnp.maximum(
            jnp.dot(t, w1_ref[...], preferred_element_type=jnp.float32) + b1_ref[...],
            0.0,
        )
        u = jnp.dot(m, w2_ref[...], preferred_element_type=jnp.float32) + b2_ref[...]
        if with_res:
            u = u + r_ref[...]
        u_ref[...] = u
        su = jnp.sum(u, axis=0, keepdims=True)
        sq = jnp.sum(u * u, axis=0, keepdims=True)
        acc = jnp.concatenate(
            [su, sq, jnp.zeros((6, u.shape[1]), jnp.float32)], axis=0
        )

        @pl.when(pl.program_id(0) == 0)
        def _():
            st_ref[...] = acc

        @pl.when(pl.program_id(0) != 0)
        def _():
            st_ref[...] = st_ref[...] + acc

    return body


_BR = 1000


def _mlp(x, agg, res, W1, b1, W2, b2):
    grid = (_N // _BR,)
    row_spec = pl.BlockSpec((_BR, _D), lambda i: (i, 0))
    full_spec = pl.BlockSpec((_D, _D), lambda i: (0, 0))
    vec_spec = pl.BlockSpec((1, _D), lambda i: (0, 0))
    st_spec = pl.BlockSpec((8, _D), lambda i: (0, 0))
    with_res = res is not None
    ins = [x, agg] + ([res] if with_res else [])
    ins += [W1, b1.reshape(1, _D), W2, b2.reshape(1, _D)]
    in_specs = [row_spec, row_spec] + ([row_spec] if with_res else [])
    in_specs += [full_spec, vec_spec, full_spec, vec_spec]
    return pl.pallas_call(
        _mk_mlp_body(with_res),
        grid=grid,
        in_specs=in_specs,
        out_specs=[row_spec, st_spec],
        out_shape=[
            jax.ShapeDtypeStruct((_N, _D), jnp.float32),
            jax.ShapeDtypeStruct((8, _D), jnp.float32),
        ],
    )(*ins)


def _bn_body(u_ref, st_ref, g_ref, b_ref, o_ref):
    inv_n = jnp.float32(1.0 / _N)
    mean = st_ref[0:1, :] * inv_n
    var = st_ref[1:2, :] * inv_n - mean * mean
    scale = g_ref[...] * lax.rsqrt(var + 1e-5)
    shift = b_ref[...] - mean * scale
    o_ref[...] = jnp.maximum(u_ref[...] * scale + shift, 0.0)


def _bn_relu(u, st, g, be):
    grid = (_N // _BR,)
    row_spec = pl.BlockSpec((_BR, _D), lambda i: (i, 0))
    st_spec = pl.BlockSpec((8, _D), lambda i: (0, 0))
    vec_spec = pl.BlockSpec((1, _D), lambda i: (0, 0))
    return pl.pallas_call(
        _bn_body,
        grid=grid,
        in_specs=[row_spec, st_spec, vec_spec, vec_spec],
        out_specs=row_spec,
        out_shape=jax.ShapeDtypeStruct((_N, _D), jnp.float32),
    )(u, st, g.reshape(1, _D), be.reshape(1, _D))


def _heads_body(h_ref, a_ref, w1m, b1m, w2m, b2m, w1l, b1l, w2l, b2l, mu_ref, lv_ref):
    t = h_ref[...] + a_ref[...]

    def head(w1, b1, w2, b2, o_ref):
        m = jnp.maximum(
            jnp.dot(t, w1[...], preferred_element_type=jnp.float32) + b1[...], 0.0
        )
        z = jnp.dot(m, w2[...], preferred_element_type=jnp.float32) + b2[...]
        o_ref[...] = 1.0 / (1.0 + jnp.exp(-z))

    head(w1m, b1m, w2m, b2m, mu_ref)
    head(w1l, b1l, w2l, b2l, lv_ref)


def _heads(h, agg, W1m, b1m, W2m, b2m, W1l, b1l, W2l, b2l):
    grid = (_N // _BR,)
    row_spec = pl.BlockSpec((_BR, _D), lambda i: (i, 0))
    full_spec = pl.BlockSpec((_D, _D), lambda i: (0, 0))
    vec_spec = pl.BlockSpec((1, _D), lambda i: (0, 0))
    return pl.pallas_call(
        _heads_body,
        grid=grid,
        in_specs=[row_spec, row_spec] + [full_spec, vec_spec] * 4,
        out_specs=[row_spec, row_spec],
        out_shape=[
            jax.ShapeDtypeStruct((_N, _D), jnp.float32),
            jax.ShapeDtypeStruct((_N, _D), jnp.float32),
        ],
    )(
        h, agg,
        W1m, b1m.reshape(1, _D), W2m, b2m.reshape(1, _D),
        W1l, b1l.reshape(1, _D), W2l, b2l.reshape(1, _D),
    )


def kernel(x, edge_index,
           W1_0, b1_0, W2_0, b2_0,
           W1_1, b1_1, W2_1, b2_1,
           W1_mu, b1_mu, W2_mu, b2_mu,
           W1_lv, b1_lv, W2_lv, b2_lv,
           g0, be0, g1, be1):
    src = edge_index[0].astype(jnp.int32)
    dst = edge_index[1].astype(jnp.int32)
    pad = jnp.full((_EP - _E,), _N, jnp.int32)
    srcp = jnp.concatenate([src, pad])
    dstp = jnp.concatenate([dst, pad]).reshape(16, _NCH, _CH)
    src2 = jnp.stack([2 * srcp, 2 * srcp + 1]).reshape(2, 16, _NCH, _CH)
    zrow = jnp.zeros((1, _D), jnp.float32)

    def conv_agg(h):
        tab = jnp.concatenate([h, zrow], axis=0).reshape(2 * (_N + 1), 128)
        return _make_segsum()(tab, src2, dstp).reshape(_N, _D)

    agg0 = conv_agg(x)
    u0, st0 = _mlp(x, agg0, None, W1_0, b1_0, W2_0, b2_0)
    h0 = _bn_relu(u0, st0, g0, be0)

    agg1 = conv_agg(h0)
    u1, st1 = _mlp(h0, agg1, h0, W1_1, b1_1, W2_1, b2_1)
    h1 = _bn_relu(u1, st1, g1, be1)

    agg2 = conv_agg(h1)
    mu, lv = _heads(h1, agg2,
                    W1_mu, b1_mu, W2_mu, b2_mu,
                    W1_lv, b1_lv, W2_lv, b2_lv)
    return (mu, lv)
